# baseline (speedup 1.0000x reference)
import jax
import jax.numpy as jnp
from jax import lax
from jax.experimental import pallas as pl
from jax.experimental.pallas import tpu as pltpu
from jax.experimental.pallas import tpu_sc as plsc

_N = 10000
_E = 320000
_D = 128
_H = 128
_DE = 16
_EPS = 1e-5

_NC = 2
_NS = 16
_NW = _NC * _NS
_EPW = _E // _NW
_EPT = _E // _NS
_ZS = 624
_ZR = 104
_CDEG = 2000
_CEG = 1000


def _deg_body(dst_hbm, out_hbm, idx_v, ones_v, zbuf_v, acc_sh):
    cid = lax.axis_index("c")
    sid = lax.axis_index("s")
    wid = sid * _NC + cid
    for j in range(_CDEG // 16):
        ones_v[pl.ds(16 * j, 16)] = jnp.ones((16,), jnp.float32)
    for j in range(_ZS // 16):
        zbuf_v[pl.ds(16 * j, 16)] = jnp.zeros((16,), jnp.float32)
    pltpu.sync_copy(zbuf_v, acc_sh.at[pl.ds(sid * _ZS, _ZS)])

    @pl.when(sid == 0)
    def _():
        pltpu.sync_copy(zbuf_v.at[pl.ds(0, 16)], acc_sh.at[pl.ds(_NS * _ZS, 16)])

    plsc.subcore_barrier()

    def chunk(k, c):
        base = wid * _EPW + k * _CDEG
        pltpu.sync_copy(dst_hbm.at[pl.ds(base, _CDEG)], idx_v)
        pltpu.sync_copy(ones_v, acc_sh.at[idx_v], add=True)
        return c

    lax.fori_loop(0, _EPW // _CDEG, chunk, 0)
    plsc.subcore_barrier()
    pltpu.sync_copy(acc_sh.at[pl.ds(sid * _ZS, _ZS)], zbuf_v)
    pltpu.sync_copy(zbuf_v, out_hbm.at[pl.ds(cid * _N + sid * _ZS, _ZS)])

    @pl.when(sid == 0)
    def _():
        pltpu.sync_copy(acc_sh.at[pl.ds(_NS * _ZS, 16)], zbuf_v.at[pl.ds(0, 16)])
        pltpu.sync_copy(zbuf_v.at[pl.ds(0, 16)],
                        out_hbm.at[pl.ds(cid * _N + _NS * _ZS, 16)])


def _make_prop_body(hdh, chp):
    def body(lo_hbm, hi_hbm, src_hbm, dst_hbm, out_lo_hbm, out_hi_hbm,
             sidx_a, didx_a, rows_a, sidx_b, didx_b, rows_b,
             zrow_v, acc_sh, sem_a, sem_b):
        cid = lax.axis_index("c")
        sid = lax.axis_index("s")

        def zfill(i, c):
            for j in range(hdh // 16):
                zrow_v[i, pl.ds(16 * j, 16)] = jnp.zeros((16,), jnp.float32)
            return c

        lax.fori_loop(0, _ZR, zfill, 0)

        def zcopy(r, c):
            pltpu.sync_copy(zrow_v, acc_sh.at[pl.ds(sid * _ZS + r * _ZR, _ZR)])
            return c

        lax.fori_loop(0, _ZS // _ZR, zcopy, 0)

        @pl.when(sid == 0)
        def _():
            pltpu.sync_copy(zrow_v.at[pl.ds(0, 16)],
                            acc_sh.at[pl.ds(_NS * _ZS, 16)])

        plsc.subcore_barrier()

        bufs = ((sidx_a, didx_a, rows_a, sem_a), (sidx_b, didx_b, rows_b, sem_b))

        def chunk(s, c):
            def do(sidx_v, didx_v, rows_v, sem):
                @pl.when(s >= 2)
                def _():
                    pltpu.make_async_copy(rows_v, acc_sh.at[didx_v], sem).wait()

                base = sid * _EPT + s * chp
                pltpu.sync_copy(src_hbm.at[pl.ds(base, chp)], sidx_v)
                pltpu.sync_copy(dst_hbm.at[pl.ds(base, chp)], didx_v)

                @pl.when(cid == 0)
                def _():
                    pltpu.sync_copy(lo_hbm.at[sidx_v], rows_v)

                @pl.when(cid == 1)
                def _():
                    pltpu.sync_copy(hi_hbm.at[sidx_v], rows_v)

                pltpu.async_copy(rows_v, acc_sh.at[didx_v], sem, add=True)

            @pl.when(s % 2 == 0)
            def _():
                do(*bufs[0])

            @pl.when(s % 2 == 1)
            def _():
                do(*bufs[1])

            return c

        lax.fori_loop(0, _EPT // chp, chunk, 0)
        pltpu.make_async_copy(rows_a, acc_sh.at[didx_a], sem_a).wait()
        pltpu.make_async_copy(rows_b, acc_sh.at[didx_b], sem_b).wait()
        plsc.subcore_barrier()

        def make_rdout(out_hbm):
            def rdout(r, c):
                pltpu.sync_copy(acc_sh.at[pl.ds(sid * _ZS + r * _ZR, _ZR)],
                                zrow_v)
                pltpu.sync_copy(
                    zrow_v, out_hbm.at[pl.ds(sid * _ZS + r * _ZR, _ZR)])
                return c
            return rdout

        def rem_out(out_hbm):
            pltpu.sync_copy(acc_sh.at[pl.ds(_NS * _ZS, 16)],
                            zrow_v.at[pl.ds(0, 16)])
            pltpu.sync_copy(zrow_v.at[pl.ds(0, 16)],
                            out_hbm.at[pl.ds(_NS * _ZS, 16)])

        @pl.when(cid == 0)
        def _():
            lax.fori_loop(0, _ZS // _ZR, make_rdout(out_lo_hbm), 0)

            @pl.when(sid == 0)
            def _():
                rem_out(out_lo_hbm)

        @pl.when(cid == 1)
        def _():
            lax.fori_loop(0, _ZS // _ZR, make_rdout(out_hi_hbm), 0)

            @pl.when(sid == 0)
            def _():
                rem_out(out_hi_hbm)

    return body


def _egather_body(h3_hbm, ilv_hbm, out_hbm,
                  idx_a, rows_a, idx_b, rows_b, sem_a, sem_b):
    cid = lax.axis_index("c")
    sid = lax.axis_index("s")
    wid = sid * _NC + cid
    ept2 = 2 * _EPW

    def step(s, c):
        base = wid * ept2 + s * _CEG

        def do(idx_v, rows_v, sem):
            @pl.when(s >= 2)
            def _():
                pltpu.make_async_copy(
                    rows_v, out_hbm.at[pl.ds(base, _CEG)], sem).wait()

            pltpu.sync_copy(ilv_hbm.at[pl.ds(base, _CEG)], idx_v)
            pltpu.sync_copy(h3_hbm.at[idx_v], rows_v)
            pltpu.async_copy(rows_v, out_hbm.at[pl.ds(base, _CEG)], sem)

        @pl.when(s % 2 == 0)
        def _():
            do(idx_a, rows_a, sem_a)

        @pl.when(s % 2 == 1)
        def _():
            do(idx_b, rows_b, sem_b)

        return c

    lax.fori_loop(0, ept2 // _CEG, step, 0)
    pltpu.make_async_copy(rows_a, out_hbm.at[pl.ds(0, _CEG)], sem_a).wait()
    pltpu.make_async_copy(rows_b, out_hbm.at[pl.ds(0, _CEG)], sem_b).wait()


_BN = 1000
_BE = 2560


def _pre_body(x_ref, dis_ref, w_ref, olo_ref, ohi_ref):
    xb = jnp.nan_to_num(x_ref[...])
    res = (jnp.dot(xb, w_ref[...], preferred_element_type=jnp.float32)
           * dis_ref[...])
    hh = res.shape[1] // 2
    olo_ref[...] = res[:, :hh]
    ohi_ref[...] = res[:, hh:]


def _mid_body(slo_ref, shi_ref, plo_ref, phi_ref, dis_ref, b_ref, g_ref,
              bt_ref, m_ref, v_ref, w_ref, olo_ref, ohi_ref):
    dis = dis_ref[...]
    y = jnp.concatenate(
        [slo_ref[...] + plo_ref[...], shi_ref[...] + phi_ref[...]], axis=1)
    y = y * dis + b_ref[...]
    t = (y - m_ref[...]) * lax.rsqrt(v_ref[...] + _EPS) * g_ref[...] + bt_ref[...]
    t = jnp.maximum(t, 0.0)
    res = (jnp.dot(t, w_ref[...], preferred_element_type=jnp.float32) * dis)
    hh = res.shape[1] // 2
    olo_ref[...] = res[:, :hh]
    ohi_ref[...] = res[:, hh:]


def _fin_body(slo_ref, shi_ref, plo_ref, phi_ref, dis_ref, b_ref, g_ref,
              bt_ref, m_ref, v_ref, o_ref):
    dis = dis_ref[...]
    y = jnp.concatenate(
        [slo_ref[...] + plo_ref[...], shi_ref[...] + phi_ref[...]], axis=1)
    y = y * dis + b_ref[...]
    t = (y - m_ref[...]) * lax.rsqrt(v_ref[...] + _EPS) * g_ref[...] + bt_ref[...]
    o_ref[...] = jnp.maximum(t, 0.0)


def _mlp_body(ef_ref, eal_ref, w1d_ref, cd_ref, b1d_ref, w2d_ref, b2d_ref,
              w3d_ref, b3t_ref, o_ref):
    eal = jnp.nan_to_num(eal_ref[...])
    z = (jnp.dot(ef_ref[...], w1d_ref[...], preferred_element_type=jnp.float32)
         + lax.dot_general(eal, cd_ref[...], (((0,), (0,)), ((), ())),
                           preferred_element_type=jnp.float32)
         + b1d_ref[...])
    z = jnp.maximum(z, 0.0)
    z = jnp.maximum(jnp.dot(z, w2d_ref[...], preferred_element_type=jnp.float32)
                    + b2d_ref[...], 0.0)
    out_t = lax.dot_general(w3d_ref[...], z, (((0,), (1,)), ((), ())),
                            preferred_element_type=jnp.float32) + b3t_ref[...]
    o_ref[...] = jnp.nan_to_num(out_t)


def _row_spec(hd):
    return pl.BlockSpec((_BN, hd), lambda i: (i, 0))


def _full_spec(r, c):
    return pl.BlockSpec((r, c), lambda i: (0, 0))


def kernel(x, edge_index, edge_attr, W1, b1, W2, b2, W3, b3,
           g1, bt1, m1, v1, g2, bt2, m2, v2, g3, bt3, m3, v3,
           eW1, eb1, eW2, eb2, eW3, eb3):
    src = edge_index[0]
    dst = edge_index[1]
    mesh = plsc.VectorSubcoreMesh(core_axis_name="c", subcore_axis_name="s")

    degp = pl.kernel(
        _deg_body,
        out_type=jax.ShapeDtypeStruct((_NC * _N,), jnp.float32),
        mesh=mesh,
        compiler_params=pltpu.CompilerParams(use_tc_tiling_on_sc=False),
        scratch_types=[
            pltpu.VMEM((_CDEG,), jnp.int32),
            pltpu.VMEM((_CDEG,), jnp.float32),
            pltpu.VMEM((_ZS,), jnp.float32),
            pltpu.VMEM_SHARED((_N,), jnp.float32),
        ],
    )(dst)
    dis_col = lax.rsqrt(degp[:_N] + degp[_N:] + 1.0)[:, None]

    def prop(hp_lo, hp_hi, hdh, chp):
        return pl.kernel(
            _make_prop_body(hdh, chp),
            out_type=[jax.ShapeDtypeStruct((_N, hdh), jnp.float32),
                      jax.ShapeDtypeStruct((_N, hdh), jnp.float32)],
            mesh=mesh,
            compiler_params=pltpu.CompilerParams(use_tc_tiling_on_sc=False),
            scratch_types=[
                pltpu.VMEM((chp,), jnp.int32),
                pltpu.VMEM((chp,), jnp.int32),
                pltpu.VMEM((chp, hdh), jnp.float32),
                pltpu.VMEM((chp,), jnp.int32),
                pltpu.VMEM((chp,), jnp.int32),
                pltpu.VMEM((chp, hdh), jnp.float32),
                pltpu.VMEM((_ZR, hdh), jnp.float32),
                pltpu.VMEM_SHARED((_N, hdh), jnp.float32),
                pltpu.SemaphoreType.DMA,
                pltpu.SemaphoreType.DMA,
            ],
        )(hp_lo, hp_hi, src, dst)

    def half_outs(hd):
        return ([_row_spec(hd // 2), _row_spec(hd // 2)],
                [jax.ShapeDtypeStruct((_N, hd // 2), jnp.float32),
                 jax.ShapeDtypeStruct((_N, hd // 2), jnp.float32)])

    o_specs, o_shapes = half_outs(_H)
    p1_lo, p1_hi = pl.pallas_call(
        _pre_body,
        grid=(_N // _BN,),
        in_specs=[_row_spec(_D), _row_spec(1), _full_spec(_D, _H)],
        out_specs=o_specs,
        out_shape=o_shapes,
    )(x, dis_col, W1)

    s1_lo, s1_hi = prop(p1_lo, p1_hi, _H // 2, 400)

    def mid(s_lo, s_hi, p_lo, p_hi, b, g, bt, m, v, W, hin, hout):
        o_specs, o_shapes = half_outs(hout)
        return pl.pallas_call(
            _mid_body,
            grid=(_N // _BN,),
            in_specs=[_row_spec(hin // 2), _row_spec(hin // 2),
                      _row_spec(hin // 2), _row_spec(hin // 2),
                      _row_spec(1),
                      _full_spec(1, hin), _full_spec(1, hin),
                      _full_spec(1, hin), _full_spec(1, hin),
                      _full_spec(1, hin), _full_spec(hin, hout)],
            out_specs=o_specs,
            out_shape=o_shapes,
        )(s_lo, s_hi, p_lo, p_hi, dis_col, b.reshape(1, -1), g.reshape(1, -1),
          bt.reshape(1, -1), m.reshape(1, -1), v.reshape(1, -1), W)

    p2_lo, p2_hi = mid(s1_lo, s1_hi, p1_lo, p1_hi, b1, g1, bt1, m1, v1,
                       W2, _H, _H // 2)
    s2_lo, s2_hi = prop(p2_lo, p2_hi, _H // 4, 1000)
    p3_lo, p3_hi = mid(s2_lo, s2_hi, p2_lo, p2_hi, b2, g2, bt2, m2, v2,
                       W3, _H // 2, _H // 4)
    s3_lo, s3_hi = prop(p3_lo, p3_hi, _H // 8, 2000)

    h3 = pl.pallas_call(
        _fin_body,
        grid=(_N // _BN,),
        in_specs=[_row_spec(_H // 8), _row_spec(_H // 8),
                  _row_spec(_H // 8), _row_spec(_H // 8),
                  _row_spec(1),
                  _full_spec(1, _H // 4), _full_spec(1, _H // 4),
                  _full_spec(1, _H // 4), _full_spec(1, _H // 4),
                  _full_spec(1, _H // 4)],
        out_specs=_row_spec(_H // 4),
        out_shape=jax.ShapeDtypeStruct((_N, _H // 4), jnp.float32),
    )(s3_lo, s3_hi, p3_lo, p3_hi, dis_col, b3.reshape(1, -1),
      g3.reshape(1, -1), bt3.reshape(1, -1), m3.reshape(1, -1),
      v3.reshape(1, -1))

    ilv = edge_index.T.reshape(2 * _E)
    efcat = pl.kernel(
        _egather_body,
        out_type=jax.ShapeDtypeStruct((2 * _E, _H // 4), jnp.float32),
        mesh=mesh,
        compiler_params=pltpu.CompilerParams(use_tc_tiling_on_sc=False),
        scratch_types=[
            pltpu.VMEM((_CEG,), jnp.int32),
            pltpu.VMEM((_CEG, _H // 4), jnp.float32),
            pltpu.VMEM((_CEG,), jnp.int32),
            pltpu.VMEM((_CEG, _H // 4), jnp.float32),
            pltpu.SemaphoreType.DMA,
            pltpu.SemaphoreType.DMA,
        ],
    )(h3, ilv)

    ef2 = efcat.reshape(_E // 2, _H)
    eal = (edge_attr.T.reshape(_DE, _E // 2, 2).transpose(2, 0, 1)
           .reshape(2 * _DE, _E // 2))
    i2 = jnp.eye(2, dtype=jnp.float32)
    w1d = jnp.kron(i2, eW1[:2 * (_H // 4), :])
    cd = jnp.kron(i2, eW1[2 * (_H // 4):, :])
    b1d = jnp.concatenate([eb1, eb1]).reshape(1, -1)
    w2d = jnp.kron(i2, eW2)
    b2d = jnp.concatenate([eb2, eb2]).reshape(1, -1)
    w3d = jnp.kron(i2, eW3)
    b3t = jnp.concatenate([eb3, eb3]).reshape(-1, 1)

    out_t = pl.pallas_call(
        _mlp_body,
        grid=(_E // _BE,),
        in_specs=[pl.BlockSpec((_BE // 2, _H), lambda i: (i, 0)),
                  pl.BlockSpec((2 * _DE, _BE // 2), lambda i: (0, i)),
                  _full_spec(_H, _H), _full_spec(2 * _DE, _H),
                  _full_spec(1, _H),
                  _full_spec(_H, _H // 2), _full_spec(1, _H // 2),
                  _full_spec(_H // 2, 4), _full_spec(4, 1)],
        out_specs=pl.BlockSpec((4, _BE // 2), lambda i: (0, i)),
        out_shape=jax.ShapeDtypeStruct((4, _E // 2), jnp.float32),
    )(ef2, eal, w1d, cd, b1d, w2d, b2d, w3d, b3t)
    return out_t.reshape(2, 2, _E // 2).transpose(2, 0, 1).reshape(_E, 2)


# speedup vs baseline: 1.1977x; 1.1977x over previous
import jax
import jax.numpy as jnp
from jax import lax
from jax.experimental import pallas as pl
from jax.experimental.pallas import tpu as pltpu
from jax.experimental.pallas import tpu_sc as plsc

_N = 10000
_E = 320000
_D = 128
_H = 128
_DE = 16
_EPS = 1e-5

_NC = 2
_NS = 16
_NW = _NC * _NS
_EPW = _E // _NW
_EPT = _E // _NS
_ZS = 624
_ZR = 104
_CDEG = 2000
_CE2 = 400


def _deg_body(dst_hbm, out_hbm, idx_v, ones_v, zbuf_v, acc_sh):
    cid = lax.axis_index("c")
    sid = lax.axis_index("s")
    wid = sid * _NC + cid
    for j in range(_CDEG // 16):
        ones_v[pl.ds(16 * j, 16)] = jnp.ones((16,), jnp.float32)
    for j in range(_ZS // 16):
        zbuf_v[pl.ds(16 * j, 16)] = jnp.zeros((16,), jnp.float32)
    pltpu.sync_copy(zbuf_v, acc_sh.at[pl.ds(sid * _ZS, _ZS)])

    @pl.when(sid == 0)
    def _():
        pltpu.sync_copy(zbuf_v.at[pl.ds(0, 16)], acc_sh.at[pl.ds(_NS * _ZS, 16)])

    plsc.subcore_barrier()

    def chunk(k, c):
        base = wid * _EPW + k * _CDEG
        pltpu.sync_copy(dst_hbm.at[pl.ds(base, _CDEG)], idx_v)
        pltpu.sync_copy(ones_v, acc_sh.at[idx_v], add=True)
        return c

    lax.fori_loop(0, _EPW // _CDEG, chunk, 0)
    plsc.subcore_barrier()
    pltpu.sync_copy(acc_sh.at[pl.ds(sid * _ZS, _ZS)], zbuf_v)
    pltpu.sync_copy(zbuf_v, out_hbm.at[pl.ds(cid * _N + sid * _ZS, _ZS)])

    @pl.when(sid == 0)
    def _():
        pltpu.sync_copy(acc_sh.at[pl.ds(_NS * _ZS, 16)], zbuf_v.at[pl.ds(0, 16)])
        pltpu.sync_copy(zbuf_v.at[pl.ds(0, 16)],
                        out_hbm.at[pl.ds(cid * _N + _NS * _ZS, 16)])


def _make_prop_body(hdh, chp):
    def body(lo_hbm, hi_hbm, src_hbm, dst_hbm, out_lo_hbm, out_hi_hbm,
             sidx_a, didx_a, rows_a, sidx_b, didx_b, rows_b,
             zrow_v, acc_sh, sem_a, sem_b):
        cid = lax.axis_index("c")
        sid = lax.axis_index("s")

        def zfill(i, c):
            for j in range(hdh // 16):
                zrow_v[i, pl.ds(16 * j, 16)] = jnp.zeros((16,), jnp.float32)
            return c

        lax.fori_loop(0, _ZR, zfill, 0)

        def zcopy(r, c):
            pltpu.sync_copy(zrow_v, acc_sh.at[pl.ds(sid * _ZS + r * _ZR, _ZR)])
            return c

        lax.fori_loop(0, _ZS // _ZR, zcopy, 0)

        @pl.when(sid == 0)
        def _():
            pltpu.sync_copy(zrow_v.at[pl.ds(0, 16)],
                            acc_sh.at[pl.ds(_NS * _ZS, 16)])

        plsc.subcore_barrier()

        bufs = ((sidx_a, didx_a, rows_a, sem_a), (sidx_b, didx_b, rows_b, sem_b))

        def chunk(s, c):
            def do(sidx_v, didx_v, rows_v, sem):
                @pl.when(s >= 2)
                def _():
                    pltpu.make_async_copy(rows_v, acc_sh.at[didx_v], sem).wait()

                base = sid * _EPT + s * chp
                pltpu.sync_copy(src_hbm.at[pl.ds(base, chp)], sidx_v)
                pltpu.sync_copy(dst_hbm.at[pl.ds(base, chp)], didx_v)

                @pl.when(cid == 0)
                def _():
                    pltpu.sync_copy(lo_hbm.at[sidx_v], rows_v)

                @pl.when(cid == 1)
                def _():
                    pltpu.sync_copy(hi_hbm.at[sidx_v], rows_v)

                pltpu.async_copy(rows_v, acc_sh.at[didx_v], sem, add=True)

            @pl.when(s % 2 == 0)
            def _():
                do(*bufs[0])

            @pl.when(s % 2 == 1)
            def _():
                do(*bufs[1])

            return c

        lax.fori_loop(0, _EPT // chp, chunk, 0)
        pltpu.make_async_copy(rows_a, acc_sh.at[didx_a], sem_a).wait()
        pltpu.make_async_copy(rows_b, acc_sh.at[didx_b], sem_b).wait()
        plsc.subcore_barrier()

        def make_rdout(out_hbm):
            def rdout(r, c):
                pltpu.sync_copy(acc_sh.at[pl.ds(sid * _ZS + r * _ZR, _ZR)],
                                zrow_v)
                pltpu.sync_copy(
                    zrow_v, out_hbm.at[pl.ds(sid * _ZS + r * _ZR, _ZR)])
                return c
            return rdout

        def rem_out(out_hbm):
            pltpu.sync_copy(acc_sh.at[pl.ds(_NS * _ZS, 16)],
                            zrow_v.at[pl.ds(0, 16)])
            pltpu.sync_copy(zrow_v.at[pl.ds(0, 16)],
                            out_hbm.at[pl.ds(_NS * _ZS, 16)])

        @pl.when(cid == 0)
        def _():
            lax.fori_loop(0, _ZS // _ZR, make_rdout(out_lo_hbm), 0)

            @pl.when(sid == 0)
            def _():
                rem_out(out_lo_hbm)

        @pl.when(cid == 1)
        def _():
            lax.fori_loop(0, _ZS // _ZR, make_rdout(out_hi_hbm), 0)

            @pl.when(sid == 0)
            def _():
                rem_out(out_hi_hbm)

    return body


def _egather_body(h3_hbm, src_hbm, dst_hbm, out_hbm,
                  sidx_a, didx_a, idx2_a, rows_a,
                  sidx_b, didx_b, idx2_b, rows_b, sem_a, sem_b):
    cid = lax.axis_index("c")
    sid = lax.axis_index("s")
    wid = sid * _NC + cid
    it2 = 2 * lax.iota(jnp.int32, 16)

    def step(s, c):
        base = wid * _EPW + s * _CE2
        obase = 2 * base

        def do(sidx_v, didx_v, idx2_v, rows_v, sem):
            @pl.when(s >= 2)
            def _():
                pltpu.make_async_copy(
                    rows_v, out_hbm.at[pl.ds(obase, 2 * _CE2)], sem).wait()

            pltpu.sync_copy(src_hbm.at[pl.ds(base, _CE2)], sidx_v)
            pltpu.sync_copy(dst_hbm.at[pl.ds(base, _CE2)], didx_v)
            for i in range(_CE2 // 16):
                pos = it2 + 32 * i
                plsc.store_scatter(idx2_v, [pos], sidx_v[pl.ds(16 * i, 16)])
                plsc.store_scatter(idx2_v, [pos + 1], didx_v[pl.ds(16 * i, 16)])
            pltpu.sync_copy(h3_hbm.at[idx2_v], rows_v)
            pltpu.async_copy(rows_v, out_hbm.at[pl.ds(obase, 2 * _CE2)], sem)

        @pl.when(s % 2 == 0)
        def _():
            do(sidx_a, didx_a, idx2_a, rows_a, sem_a)

        @pl.when(s % 2 == 1)
        def _():
            do(sidx_b, didx_b, idx2_b, rows_b, sem_b)

        return c

    lax.fori_loop(0, _EPW // _CE2, step, 0)
    pltpu.make_async_copy(rows_a, out_hbm.at[pl.ds(0, 2 * _CE2)], sem_a).wait()
    pltpu.make_async_copy(rows_b, out_hbm.at[pl.ds(0, 2 * _CE2)], sem_b).wait()


_BN = 1000
_BE = 2560


def _pre_body(x_ref, dis_ref, w_ref, olo_ref, ohi_ref):
    xb = jnp.nan_to_num(x_ref[...])
    res = (jnp.dot(xb, w_ref[...], preferred_element_type=jnp.float32)
           * dis_ref[...])
    hh = res.shape[1] // 2
    olo_ref[...] = res[:, :hh]
    ohi_ref[...] = res[:, hh:]


def _mid_body(slo_ref, shi_ref, plo_ref, phi_ref, dis_ref, b_ref, g_ref,
              bt_ref, m_ref, v_ref, w_ref, olo_ref, ohi_ref):
    dis = dis_ref[...]
    y = jnp.concatenate(
        [slo_ref[...] + plo_ref[...], shi_ref[...] + phi_ref[...]], axis=1)
    y = y * dis + b_ref[...]
    t = (y - m_ref[...]) * lax.rsqrt(v_ref[...] + _EPS) * g_ref[...] + bt_ref[...]
    t = jnp.maximum(t, 0.0)
    res = (jnp.dot(t, w_ref[...], preferred_element_type=jnp.float32) * dis)
    hh = res.shape[1] // 2
    olo_ref[...] = res[:, :hh]
    ohi_ref[...] = res[:, hh:]


def _fin_body(slo_ref, shi_ref, plo_ref, phi_ref, dis_ref, b_ref, g_ref,
              bt_ref, m_ref, v_ref, o_ref):
    dis = dis_ref[...]
    y = jnp.concatenate(
        [slo_ref[...] + plo_ref[...], shi_ref[...] + phi_ref[...]], axis=1)
    y = y * dis + b_ref[...]
    t = (y - m_ref[...]) * lax.rsqrt(v_ref[...] + _EPS) * g_ref[...] + bt_ref[...]
    o_ref[...] = jnp.maximum(t, 0.0)


def _mlp_body(ef_ref, eal_ref, w1d_ref, cd_ref, b1d_ref, w2d_ref, b2d_ref,
              w3d_ref, b3t_ref, o_ref):
    eal = jnp.nan_to_num(eal_ref[...])
    z = (jnp.dot(ef_ref[...], w1d_ref[...], preferred_element_type=jnp.float32)
         + lax.dot_general(eal, cd_ref[...], (((0,), (0,)), ((), ())),
                           preferred_element_type=jnp.float32)
         + b1d_ref[...])
    z = jnp.maximum(z, 0.0)
    z = jnp.maximum(jnp.dot(z, w2d_ref[...], preferred_element_type=jnp.float32)
                    + b2d_ref[...], 0.0)
    out_p = (jnp.dot(z, w3d_ref[...], preferred_element_type=jnp.float32)
             + b3t_ref[...])
    o_ref[...] = jnp.nan_to_num(out_p)


def _row_spec(hd):
    return pl.BlockSpec((_BN, hd), lambda i: (i, 0))


def _full_spec(r, c):
    return pl.BlockSpec((r, c), lambda i: (0, 0))


def kernel(x, edge_index, edge_attr, W1, b1, W2, b2, W3, b3,
           g1, bt1, m1, v1, g2, bt2, m2, v2, g3, bt3, m3, v3,
           eW1, eb1, eW2, eb2, eW3, eb3):
    src = edge_index[0]
    dst = edge_index[1]
    mesh = plsc.VectorSubcoreMesh(core_axis_name="c", subcore_axis_name="s")

    degp = pl.kernel(
        _deg_body,
        out_type=jax.ShapeDtypeStruct((_NC * _N,), jnp.float32),
        mesh=mesh,
        compiler_params=pltpu.CompilerParams(use_tc_tiling_on_sc=False),
        scratch_types=[
            pltpu.VMEM((_CDEG,), jnp.int32),
            pltpu.VMEM((_CDEG,), jnp.float32),
            pltpu.VMEM((_ZS,), jnp.float32),
            pltpu.VMEM_SHARED((_N,), jnp.float32),
        ],
    )(dst)
    dis_col = lax.rsqrt(degp[:_N] + degp[_N:] + 1.0)[:, None]

    def prop(hp_lo, hp_hi, hdh, chp):
        return pl.kernel(
            _make_prop_body(hdh, chp),
            out_type=[jax.ShapeDtypeStruct((_N, hdh), jnp.float32),
                      jax.ShapeDtypeStruct((_N, hdh), jnp.float32)],
            mesh=mesh,
            compiler_params=pltpu.CompilerParams(use_tc_tiling_on_sc=False),
            scratch_types=[
                pltpu.VMEM((chp,), jnp.int32),
                pltpu.VMEM((chp,), jnp.int32),
                pltpu.VMEM((chp, hdh), jnp.float32),
                pltpu.VMEM((chp,), jnp.int32),
                pltpu.VMEM((chp,), jnp.int32),
                pltpu.VMEM((chp, hdh), jnp.float32),
                pltpu.VMEM((_ZR, hdh), jnp.float32),
                pltpu.VMEM_SHARED((_N, hdh), jnp.float32),
                pltpu.SemaphoreType.DMA,
                pltpu.SemaphoreType.DMA,
            ],
        )(hp_lo, hp_hi, src, dst)

    def half_outs(hd):
        return ([_row_spec(hd // 2), _row_spec(hd // 2)],
                [jax.ShapeDtypeStruct((_N, hd // 2), jnp.float32),
                 jax.ShapeDtypeStruct((_N, hd // 2), jnp.float32)])

    o_specs, o_shapes = half_outs(_H)
    p1_lo, p1_hi = pl.pallas_call(
        _pre_body,
        grid=(_N // _BN,),
        in_specs=[_row_spec(_D), _row_spec(1), _full_spec(_D, _H)],
        out_specs=o_specs,
        out_shape=o_shapes,
    )(x, dis_col, W1)

    s1_lo, s1_hi = prop(p1_lo, p1_hi, _H // 2, 400)

    def mid(s_lo, s_hi, p_lo, p_hi, b, g, bt, m, v, W, hin, hout):
        o_specs, o_shapes = half_outs(hout)
        return pl.pallas_call(
            _mid_body,
            grid=(_N // _BN,),
            in_specs=[_row_spec(hin // 2), _row_spec(hin // 2),
                      _row_spec(hin // 2), _row_spec(hin // 2),
                      _row_spec(1),
                      _full_spec(1, hin), _full_spec(1, hin),
                      _full_spec(1, hin), _full_spec(1, hin),
                      _full_spec(1, hin), _full_spec(hin, hout)],
            out_specs=o_specs,
            out_shape=o_shapes,
        )(s_lo, s_hi, p_lo, p_hi, dis_col, b.reshape(1, -1), g.reshape(1, -1),
          bt.reshape(1, -1), m.reshape(1, -1), v.reshape(1, -1), W)

    p2_lo, p2_hi = mid(s1_lo, s1_hi, p1_lo, p1_hi, b1, g1, bt1, m1, v1,
                       W2, _H, _H // 2)
    s2_lo, s2_hi = prop(p2_lo, p2_hi, _H // 4, 1000)
    p3_lo, p3_hi = mid(s2_lo, s2_hi, p2_lo, p2_hi, b2, g2, bt2, m2, v2,
                       W3, _H // 2, _H // 4)
    s3_lo, s3_hi = prop(p3_lo, p3_hi, _H // 8, 2000)

    h3 = pl.pallas_call(
        _fin_body,
        grid=(_N // _BN,),
        in_specs=[_row_spec(_H // 8), _row_spec(_H // 8),
                  _row_spec(_H // 8), _row_spec(_H // 8),
                  _row_spec(1),
                  _full_spec(1, _H // 4), _full_spec(1, _H // 4),
                  _full_spec(1, _H // 4), _full_spec(1, _H // 4),
                  _full_spec(1, _H // 4)],
        out_specs=_row_spec(_H // 4),
        out_shape=jax.ShapeDtypeStruct((_N, _H // 4), jnp.float32),
    )(s3_lo, s3_hi, p3_lo, p3_hi, dis_col, b3.reshape(1, -1),
      g3.reshape(1, -1), bt3.reshape(1, -1), m3.reshape(1, -1),
      v3.reshape(1, -1))

    efcat = pl.kernel(
        _egather_body,
        out_type=jax.ShapeDtypeStruct((2 * _E, _H // 4), jnp.float32),
        mesh=mesh,
        compiler_params=pltpu.CompilerParams(use_tc_tiling_on_sc=False,
                                             needs_layout_passes=False),
        scratch_types=[
            pltpu.VMEM((_CE2,), jnp.int32),
            pltpu.VMEM((_CE2,), jnp.int32),
            pltpu.VMEM((2 * _CE2,), jnp.int32),
            pltpu.VMEM((2 * _CE2, _H // 4), jnp.float32),
            pltpu.VMEM((_CE2,), jnp.int32),
            pltpu.VMEM((_CE2,), jnp.int32),
            pltpu.VMEM((2 * _CE2,), jnp.int32),
            pltpu.VMEM((2 * _CE2, _H // 4), jnp.float32),
            pltpu.SemaphoreType.DMA,
            pltpu.SemaphoreType.DMA,
        ],
    )(h3, src, dst)

    ef2 = efcat.reshape(_E // 2, _H)
    eal = (edge_attr.T.reshape(_DE, _E // 2, 2).transpose(2, 0, 1)
           .reshape(2 * _DE, _E // 2))
    i2 = jnp.eye(2, dtype=jnp.float32)
    w1d = jnp.kron(i2, eW1[:2 * (_H // 4), :])
    cd = jnp.kron(i2, eW1[2 * (_H // 4):, :])
    b1d = jnp.concatenate([eb1, eb1]).reshape(1, -1)
    w2d = jnp.kron(i2, eW2)
    b2d = jnp.concatenate([eb2, eb2]).reshape(1, -1)
    w3d = jnp.kron(i2, eW3)
    b3t = jnp.concatenate([eb3, eb3]).reshape(1, -1)

    out_t = pl.pallas_call(
        _mlp_body,
        grid=(_E // _BE,),
        in_specs=[pl.BlockSpec((_BE // 2, _H), lambda i: (i, 0)),
                  pl.BlockSpec((2 * _DE, _BE // 2), lambda i: (0, i)),
                  _full_spec(_H, _H), _full_spec(2 * _DE, _H),
                  _full_spec(1, _H),
                  _full_spec(_H, _H // 2), _full_spec(1, _H // 2),
                  _full_spec(_H // 2, 4), _full_spec(1, 4)],
        out_specs=pl.BlockSpec((_BE // 2, 4), lambda i: (i, 0)),
        out_shape=jax.ShapeDtypeStruct((_E // 2, 4), jnp.float32),
    )(ef2, eal, w1d, cd, b1d, w2d, b2d, w3d, b3t)
    return out_t.reshape(_E, 2)
